# Initial kernel scaffold; baseline (speedup 1.0000x reference)
import jax
import jax.numpy as jnp
from jax.experimental import pallas as pl


def kernel(x):
    raise NotImplementedError("write your pallas kernel here")


# speedup vs baseline: 1.3034x; 1.3034x over previous
import jax
import jax.numpy as jnp
from jax import lax
from jax.experimental import pallas as pl
from jax.experimental.pallas import tpu as pltpu
from jax.experimental.pallas import tpu_sc as plsc

_COLS = (0, 1, 5)
_LEAD = 16


def _make_sc_kernel(n_rows: int, dtype):
    info = plsc.get_sparse_core_info()
    nc, ns, lanes = info.num_cores, info.num_subcores, info.num_lanes
    nw = nc * ns
    chunk = n_rows // nw
    assert chunk % lanes == 0 and chunk % 8 == 0

    def body(x_hbm, out_hbm, rows_v, out_v):
        wid = lax.axis_index("s") * nc + lax.axis_index("c")
        base = wid * chunk
        pltpu.sync_copy(x_hbm.at[pl.ds(base, chunk), pl.ds(0, _LEAD)], rows_v)
        lane_iota = lax.iota(jnp.int32, lanes)
        for i, c in enumerate(_COLS):
            cidx = jnp.full((lanes,), c, jnp.int32)
            for t in range(chunk // lanes):
                ridx = lane_iota + t * lanes
                vals = plsc.load_gather(rows_v, [ridx, cidx])
                out_v[i, pl.ds(t * lanes, lanes)] = vals
        pltpu.sync_copy(out_v, out_hbm.at[:, pl.ds(base, chunk)])

    return pl.kernel(
        body,
        out_type=jax.ShapeDtypeStruct((len(_COLS), n_rows), dtype),
        mesh=plsc.VectorSubcoreMesh(core_axis_name="c", subcore_axis_name="s"),
        scratch_types=[
            pltpu.VMEM((chunk, _LEAD), jnp.float32),
            pltpu.VMEM((len(_COLS), chunk), jnp.float32),
        ],
        compiler_params=pltpu.CompilerParams(
            use_tc_tiling_on_sc=False, needs_layout_passes=False
        ),
    )


def kernel(x):
    n_rows = x.shape[0]
    return _make_sc_kernel(n_rows, x.dtype)(x)


# speedup vs baseline: 1.3086x; 1.0040x over previous
import jax
import jax.numpy as jnp
from jax import lax
from jax.experimental import pallas as pl
from jax.experimental.pallas import tpu as pltpu
from jax.experimental.pallas import tpu_sc as plsc

_COLS = (0, 1, 5)
_LEAD = 16


def _make_sc_kernel(n_rows: int, dtype):
    info = plsc.get_sparse_core_info()
    nc, ns, lanes = info.num_cores, info.num_subcores, info.num_lanes
    nw = nc * ns
    chunk = n_rows // nw
    assert chunk % lanes == 0 and chunk % 8 == 0

    half = chunk // 2

    def body(x_hbm, out_hbm, rows_v, out_v, sem_in, sem_out):
        wid = lax.axis_index("s") * nc + lax.axis_index("c")
        base = wid * chunk
        stage = [
            pltpu.make_async_copy(
                x_hbm.at[pl.ds(base + h * half, half), pl.ds(0, _LEAD)],
                rows_v.at[pl.ds(h * half, half)],
                sem_in,
            )
            for h in range(2)
        ]
        stage[0].start()
        stage[1].start()
        lane_iota = lax.iota(jnp.int32, lanes)

        def extract(h):
            for i, c in enumerate(_COLS):
                cidx = jnp.full((lanes,), c, jnp.int32)
                for t in range(h * half // lanes, (h + 1) * half // lanes):
                    ridx = lane_iota + t * lanes
                    vals = plsc.load_gather(rows_v, [ridx, cidx])
                    out_v[i, pl.ds(t * lanes, lanes)] = vals

        drain = [
            pltpu.make_async_copy(
                out_v.at[:, pl.ds(h * half, half)],
                out_hbm.at[:, pl.ds(base + h * half, half)],
                sem_out,
            )
            for h in range(2)
        ]
        stage[0].wait()
        extract(0)
        drain[0].start()
        stage[1].wait()
        extract(1)
        drain[1].start()
        drain[0].wait()
        drain[1].wait()

    return pl.kernel(
        body,
        out_type=jax.ShapeDtypeStruct((len(_COLS), n_rows), dtype),
        mesh=plsc.VectorSubcoreMesh(core_axis_name="c", subcore_axis_name="s"),
        scratch_types=[
            pltpu.VMEM((chunk, _LEAD), jnp.float32),
            pltpu.VMEM((len(_COLS), chunk), jnp.float32),
            pltpu.SemaphoreType.DMA,
            pltpu.SemaphoreType.DMA,
        ],
        compiler_params=pltpu.CompilerParams(
            use_tc_tiling_on_sc=False, needs_layout_passes=False
        ),
    )


def kernel(x):
    n_rows = x.shape[0]
    return _make_sc_kernel(n_rows, x.dtype)(x)


# speedup vs baseline: 1.3237x; 1.0115x over previous
import jax
import jax.numpy as jnp
from jax import lax
from jax.experimental import pallas as pl
from jax.experimental.pallas import tpu as pltpu
from jax.experimental.pallas import tpu_sc as plsc

_COLS = (0, 1, 5)
_LEAD = 16


def _make_sc_kernel(n_rows: int, dtype):
    info = plsc.get_sparse_core_info()
    nc, ns, lanes = info.num_cores, info.num_subcores, info.num_lanes
    nw = nc * ns
    chunk = n_rows // nw
    assert chunk % lanes == 0 and chunk % 8 == 0

    half = chunk // 2

    def body(x_hbm, out_hbm, rows_v, out_v, sem_in, sem_out):
        wid = lax.axis_index("s") * nc + lax.axis_index("c")
        base = wid * chunk
        stage = [
            pltpu.make_async_copy(
                x_hbm.at[pl.ds(base + h * half, half), pl.ds(0, _LEAD)],
                rows_v.at[pl.ds(h * half, half)],
                sem_in,
            )
            for h in range(2)
        ]
        stage[0].start()
        stage[1].start()
        lane_iota = lax.iota(jnp.int32, lanes)

        cidxs = [jnp.full((lanes,), c, jnp.int32) for c in _COLS]

        def extract(h):
            group = 8
            for t0 in range(h * half // lanes, (h + 1) * half // lanes, group):
                batch = []
                for i in range(len(_COLS)):
                    for t in range(t0, t0 + group):
                        ridx = lane_iota + t * lanes
                        batch.append(
                            (i, t, plsc.load_gather(rows_v, [ridx, cidxs[i]]))
                        )
                for i, t, vals in batch:
                    out_v[i, pl.ds(t * lanes, lanes)] = vals

        drain = [
            pltpu.make_async_copy(
                out_v.at[:, pl.ds(h * half, half)],
                out_hbm.at[:, pl.ds(base + h * half, half)],
                sem_out,
            )
            for h in range(2)
        ]
        stage[0].wait()
        extract(0)
        drain[0].start()
        stage[1].wait()
        extract(1)
        drain[1].start()
        drain[0].wait()
        drain[1].wait()

    return pl.kernel(
        body,
        out_type=jax.ShapeDtypeStruct((len(_COLS), n_rows), dtype),
        mesh=plsc.VectorSubcoreMesh(core_axis_name="c", subcore_axis_name="s"),
        scratch_types=[
            pltpu.VMEM((chunk, _LEAD), jnp.float32),
            pltpu.VMEM((len(_COLS), chunk), jnp.float32),
            pltpu.SemaphoreType.DMA,
            pltpu.SemaphoreType.DMA,
        ],
        compiler_params=pltpu.CompilerParams(
            use_tc_tiling_on_sc=False, needs_layout_passes=False
        ),
    )


def kernel(x):
    n_rows = x.shape[0]
    return _make_sc_kernel(n_rows, x.dtype)(x)


# speedup vs baseline: 1.3299x; 1.0047x over previous
import jax
import jax.numpy as jnp
from jax import lax
from jax.experimental import pallas as pl
from jax.experimental.pallas import tpu as pltpu
from jax.experimental.pallas import tpu_sc as plsc

_COLS = (0, 1, 5)
_LEAD = 16


def _make_sc_kernel(n_rows: int, dtype):
    info = plsc.get_sparse_core_info()
    nc, ns, lanes = info.num_cores, info.num_subcores, info.num_lanes
    nw = nc * ns
    chunk = n_rows // nw
    assert chunk % lanes == 0 and chunk % 8 == 0

    n_chunks = 4
    piece = chunk // n_chunks

    def body(x_hbm, out_hbm, rows_v, out_v, sem_in, sem_out):
        wid = lax.axis_index("s") * nc + lax.axis_index("c")
        base = wid * chunk
        stage = [
            pltpu.make_async_copy(
                x_hbm.at[pl.ds(base + h * piece, piece), pl.ds(0, _LEAD)],
                rows_v.at[pl.ds(h * piece, piece)],
                sem_in,
            )
            for h in range(n_chunks)
        ]
        for cp in stage:
            cp.start()
        lane_iota = lax.iota(jnp.int32, lanes)

        cidxs = [jnp.full((lanes,), c, jnp.int32) for c in _COLS]

        def extract(h):
            group = 8
            for t0 in range(h * piece // lanes, (h + 1) * piece // lanes, group):
                batch = []
                for i in range(len(_COLS)):
                    for t in range(t0, t0 + group):
                        ridx = lane_iota + t * lanes
                        batch.append(
                            (i, t, plsc.load_gather(rows_v, [ridx, cidxs[i]]))
                        )
                for i, t, vals in batch:
                    out_v[i, pl.ds(t * lanes, lanes)] = vals

        drain = [
            pltpu.make_async_copy(
                out_v.at[:, pl.ds(h * piece, piece)],
                out_hbm.at[:, pl.ds(base + h * piece, piece)],
                sem_out,
            )
            for h in range(n_chunks)
        ]
        for h in range(n_chunks):
            stage[h].wait()
            extract(h)
            drain[h].start()
        for h in range(n_chunks):
            drain[h].wait()

    return pl.kernel(
        body,
        out_type=jax.ShapeDtypeStruct((len(_COLS), n_rows), dtype),
        mesh=plsc.VectorSubcoreMesh(core_axis_name="c", subcore_axis_name="s"),
        scratch_types=[
            pltpu.VMEM((chunk, _LEAD), jnp.float32),
            pltpu.VMEM((len(_COLS), chunk), jnp.float32),
            pltpu.SemaphoreType.DMA,
            pltpu.SemaphoreType.DMA,
        ],
        compiler_params=pltpu.CompilerParams(
            use_tc_tiling_on_sc=False, needs_layout_passes=False
        ),
    )


def kernel(x):
    n_rows = x.shape[0]
    return _make_sc_kernel(n_rows, x.dtype)(x)
